# baseline (speedup 1.0000x reference)
import functools

import jax
import jax.numpy as jnp
from jax import lax
from jax.experimental import pallas as pl
from jax.experimental.pallas import tpu as pltpu
from jax.experimental.pallas import tpu_sc as plsc

BATCH = 16384
EMBED = 128
_CHUNKROWS = 128
_NW = 32
_NSPLIT = 2
_BS = BATCH // _NSPLIT
_G = _BS // _CHUNKROWS
_CPW = _G // _NW


def _sc_gather_build():
    mesh = plsc.VectorSubcoreMesh(core_axis_name="c", subcore_axis_name="s")

    @functools.partial(
        pl.kernel,
        mesh=mesh,
        out_type=(
            jax.ShapeDtypeStruct((_G, _CHUNKROWS, EMBED), jnp.float32),
            jax.ShapeDtypeStruct((_G, _CHUNKROWS, EMBED), jnp.float32),
        ),
        scratch_types=[
            pltpu.VMEM((_CPW, _CHUNKROWS), jnp.int32),
            pltpu.VMEM((_CPW, _CHUNKROWS), jnp.int32),
            pltpu.VMEM((_CPW, _CHUNKROWS, EMBED), jnp.float32),
            pltpu.VMEM((_CPW, _CHUNKROWS, EMBED), jnp.float32),
            pltpu.SemaphoreType.DMA,
            pltpu.SemaphoreType.DMA,
        ],
        compiler_params=pltpu.CompilerParams(use_tc_tiling_on_sc=True),
    )
    def sc_gather(users2d, items2d, eu_table, ei_table, eu_out, ei_out,
                  uidx_v, iidx_v, urows_v, irows_v, sem_u, sem_i):
        wid = lax.axis_index("s") * 2 + lax.axis_index("c")
        src = wid * _CPW
        dst = wid * _CPW

        pltpu.sync_copy(users2d.at[pl.ds(src, _CPW)], uidx_v)
        pltpu.sync_copy(items2d.at[pl.ds(src, _CPW)], iidx_v)
        ucps = [
            pltpu.make_async_copy(eu_table.at[uidx_v.at[j]], urows_v.at[j],
                                  sem_u)
            for j in range(_CPW)
        ]
        icps = [
            pltpu.make_async_copy(ei_table.at[iidx_v.at[j]], irows_v.at[j],
                                  sem_i)
            for j in range(_CPW)
        ]
        for c in ucps:
            c.start()
        for c in icps:
            c.start()
        for c in ucps:
            c.wait()
        pltpu.sync_copy(urows_v, eu_out.at[pl.ds(dst, _CPW)])
        for c in icps:
            c.wait()
        pltpu.sync_copy(irows_v, ei_out.at[pl.ds(dst, _CPW)])

    return sc_gather


_sc_gather_cache = None


def _sc_gather(*args):
    global _sc_gather_cache
    if _sc_gather_cache is None:
        _sc_gather_cache = _sc_gather_build()
    return _sc_gather_cache(*args)


def _mlp_body(eu_ref, ei_ref, w0u_ref, w0i_ref, b0_ref, w1_ref, b1_ref,
              w2_ref, b2_ref, wout_ref, bout_ref, out_ref):
    h = eu_ref[...] @ w0u_ref[...] + ei_ref[...] @ w0i_ref[...] + b0_ref[...]
    h = jnp.maximum(h, 0.0)
    h = jnp.maximum(h @ w1_ref[...] + b1_ref[...], 0.0)
    h = jnp.maximum(h @ w2_ref[...] + b2_ref[...], 0.0)
    s = h @ wout_ref[...] + bout_ref[0, 0]
    out_ref[...] = s.reshape(1, 1, -1)


def _mlp(eu, ei, w0u, w0i, b0, w1, b1, w2, b2, wout, bout, blk=4096):
    rows = eu.shape[0]
    nblk = rows // blk
    full = lambda i: (0, 0)
    return pl.pallas_call(
        _mlp_body,
        grid=(nblk,),
        in_specs=[
            pl.BlockSpec((blk, EMBED), lambda i: (i, 0)),
            pl.BlockSpec((blk, EMBED), lambda i: (i, 0)),
            pl.BlockSpec((128, 128), full),
            pl.BlockSpec((128, 128), full),
            pl.BlockSpec((1, 128), full),
            pl.BlockSpec((128, 64), full),
            pl.BlockSpec((1, 64), full),
            pl.BlockSpec((64, 32), full),
            pl.BlockSpec((1, 32), full),
            pl.BlockSpec((32, 1), full),
            pl.BlockSpec((1, 1), full),
        ],
        out_specs=pl.BlockSpec((1, 1, blk), lambda i: (i, 0, 0)),
        out_shape=jax.ShapeDtypeStruct((nblk, 1, blk), jnp.float32),
    )(eu, ei, w0u, w0i, b0, w1, b1, w2, b2, wout, bout)


def kernel(users, items, embed_user, embed_item, W0, b0, W1, b1, W2, b2,
           Wout, bout):
    users2d = users.astype(jnp.int32).reshape(BATCH // _CHUNKROWS, _CHUNKROWS)
    items2d = items.astype(jnp.int32).reshape(BATCH // _CHUNKROWS, _CHUNKROWS)
    w0u = W0[:, :EMBED].T
    w0i = W0[:, EMBED:].T
    b0r = b0.reshape(1, -1)
    w1t, b1r = W1.T, b1.reshape(1, -1)
    w2t, b2r = W2.T, b2.reshape(1, -1)
    woutt, boutr = Wout.T, bout.reshape(1, 1)

    outs = []
    for c in range(_NSPLIT):
        eu3, ei3 = _sc_gather(
            lax.slice_in_dim(users2d, c * _G, (c + 1) * _G, axis=0),
            lax.slice_in_dim(items2d, c * _G, (c + 1) * _G, axis=0),
            embed_user, embed_item)
        outs.append(_mlp(eu3.reshape(_BS, EMBED), ei3.reshape(_BS, EMBED),
                         w0u, w0i, b0r, w1t, b1r, w2t, b2r, woutt, boutr))
    out = outs[0] if _NSPLIT == 1 else jnp.concatenate(outs, axis=0)
    return out.reshape(-1)


# speedup vs baseline: 1.0052x; 1.0052x over previous
import functools

import jax
import jax.numpy as jnp
from jax import lax
from jax.experimental import pallas as pl
from jax.experimental.pallas import tpu as pltpu
from jax.experimental.pallas import tpu_sc as plsc

BATCH = 16384
EMBED = 128
_CHUNKROWS = 128
_NW = 32
_NSPLIT = 2
_BS = BATCH // _NSPLIT
_G = _BS // _CHUNKROWS
_CPW = _G // _NW


def _sc_gather_build(chunk):
    mesh = plsc.VectorSubcoreMesh(core_axis_name="c", subcore_axis_name="s")

    @functools.partial(
        pl.kernel,
        mesh=mesh,
        out_type=(
            jax.ShapeDtypeStruct((_G, _CHUNKROWS, EMBED), jnp.float32),
            jax.ShapeDtypeStruct((_G, _CHUNKROWS, EMBED), jnp.float32),
        ),
        scratch_types=[
            pltpu.VMEM((_CPW, _CHUNKROWS), jnp.int32),
            pltpu.VMEM((_CPW, _CHUNKROWS), jnp.int32),
            pltpu.VMEM((_CPW, _CHUNKROWS, EMBED), jnp.float32),
            pltpu.VMEM((_CPW, _CHUNKROWS, EMBED), jnp.float32),
            pltpu.SemaphoreType.DMA,
            pltpu.SemaphoreType.DMA,
        ],
        compiler_params=pltpu.CompilerParams(use_tc_tiling_on_sc=True),
    )
    def sc_gather(users2d, items2d, eu_table, ei_table, eu_out, ei_out,
                  uidx_v, iidx_v, urows_v, irows_v, sem_u, sem_i):
        wid = lax.axis_index("s") * 2 + lax.axis_index("c")
        src = chunk * _G + wid * _CPW
        dst = wid * _CPW

        pltpu.sync_copy(users2d.at[pl.ds(src, _CPW)], uidx_v)
        pltpu.sync_copy(items2d.at[pl.ds(src, _CPW)], iidx_v)
        ucps = [
            pltpu.make_async_copy(eu_table.at[uidx_v.at[j]], urows_v.at[j],
                                  sem_u)
            for j in range(_CPW)
        ]
        icps = [
            pltpu.make_async_copy(ei_table.at[iidx_v.at[j]], irows_v.at[j],
                                  sem_i)
            for j in range(_CPW)
        ]
        for c in ucps:
            c.start()
        for c in icps:
            c.start()
        for c in ucps:
            c.wait()
        pltpu.sync_copy(urows_v, eu_out.at[pl.ds(dst, _CPW)])
        for c in icps:
            c.wait()
        pltpu.sync_copy(irows_v, ei_out.at[pl.ds(dst, _CPW)])

    return sc_gather


_sc_gather_cache = {}


def _sc_gather(chunk, *args):
    if chunk not in _sc_gather_cache:
        _sc_gather_cache[chunk] = _sc_gather_build(chunk)
    return _sc_gather_cache[chunk](*args)


def _mlp_body(eu_ref, ei_ref, w0u_ref, w0i_ref, b0_ref, w1_ref, b1_ref,
              w2_ref, b2_ref, wout_ref, bout_ref, out_ref):
    h = eu_ref[...] @ w0u_ref[...] + ei_ref[...] @ w0i_ref[...] + b0_ref[...]
    h = jnp.maximum(h, 0.0)
    h = jnp.maximum(h @ w1_ref[...] + b1_ref[...], 0.0)
    h = jnp.maximum(h @ w2_ref[...] + b2_ref[...], 0.0)
    s = h @ wout_ref[...] + bout_ref[0, 0]
    out_ref[...] = s.reshape(1, 1, -1)


def _mlp(eu, ei, w0u, w0i, b0, w1, b1, w2, b2, wout, bout, blk=4096):
    rows = eu.shape[0]
    nblk = rows // blk
    full = lambda i: (0, 0)
    return pl.pallas_call(
        _mlp_body,
        grid=(nblk,),
        in_specs=[
            pl.BlockSpec((blk, EMBED), lambda i: (i, 0)),
            pl.BlockSpec((blk, EMBED), lambda i: (i, 0)),
            pl.BlockSpec((128, 128), full),
            pl.BlockSpec((128, 128), full),
            pl.BlockSpec((1, 128), full),
            pl.BlockSpec((128, 64), full),
            pl.BlockSpec((1, 64), full),
            pl.BlockSpec((64, 32), full),
            pl.BlockSpec((1, 32), full),
            pl.BlockSpec((32, 1), full),
            pl.BlockSpec((1, 1), full),
        ],
        out_specs=pl.BlockSpec((1, 1, blk), lambda i: (i, 0, 0)),
        out_shape=jax.ShapeDtypeStruct((nblk, 1, blk), jnp.float32),
    )(eu, ei, w0u, w0i, b0, w1, b1, w2, b2, wout, bout)


def kernel(users, items, embed_user, embed_item, W0, b0, W1, b1, W2, b2,
           Wout, bout):
    users2d = users.astype(jnp.int32).reshape(BATCH // _CHUNKROWS, _CHUNKROWS)
    items2d = items.astype(jnp.int32).reshape(BATCH // _CHUNKROWS, _CHUNKROWS)
    w0u = W0[:, :EMBED].T
    w0i = W0[:, EMBED:].T
    b0r = b0.reshape(1, -1)
    w1t, b1r = W1.T, b1.reshape(1, -1)
    w2t, b2r = W2.T, b2.reshape(1, -1)
    woutt, boutr = Wout.T, bout.reshape(1, 1)

    outs = []
    for c in range(_NSPLIT):
        eu3, ei3 = _sc_gather(c, users2d, items2d, embed_user, embed_item)
        outs.append(_mlp(eu3.reshape(_BS, EMBED), ei3.reshape(_BS, EMBED),
                         w0u, w0i, b0r, w1t, b1r, w2t, b2r, woutt, boutr))
    out = outs[0] if _NSPLIT == 1 else jnp.concatenate(outs, axis=0)
    return out.reshape(-1)
